# Initial kernel scaffold; baseline (speedup 1.0000x reference)
import jax
import jax.numpy as jnp
from jax.experimental import pallas as pl


def kernel(x, edge_index_imports, edge_index_references, Wl0_imports, bl0_imports, Wr0_imports, Wl1_imports, bl1_imports, Wr1_imports, Wl0_references, bl0_references, Wr0_references, Wl1_references, bl1_references, Wr1_references, edge_type_attention):
    raise NotImplementedError("write your pallas kernel here")


# speedup vs baseline: 3.7979x; 3.7979x over previous
import functools

import jax
import jax.numpy as jnp
from jax import lax
from jax.experimental import pallas as pl
from jax.experimental.pallas import tpu as pltpu
from jax.experimental.pallas import tpu_sc as plsc

N = 10000
E = 320000
D = 128
WD = 144

NC = 2
NS = 16
C = 96
EPT = 20064
EPAD = EPT * NS
CHUNKS = EPT // C
NPAD = 10112
RPT = NPAD // NS
PADROW = 10048

BLK = 400


_PIECES = []
_off = 0
while _off < RPT:
    _PIECES.append((_off, min(C, RPT - _off)))
    _off += C


def _seg_body(table, src, dst, z2d,
              sums_out,
              src_v, dst_v, rows_v, acc, sem):
    c = lax.axis_index("c")
    s = lax.axis_index("s")
    row0 = s * RPT
    pltpu.sync_copy(z2d, rows_v)
    for off, sz in _PIECES:
        pltpu.sync_copy(rows_v.at[pl.ds(0, sz)], acc.at[pl.ds(row0 + off, sz)])
    plsc.subcore_barrier()

    ebase = s * EPT

    def chunk(i, carry):
        base = c * EPAD + ebase + i * C
        pltpu.sync_copy(src.at[pl.ds(base, C)], src_v)
        pltpu.sync_copy(dst.at[pl.ds(base, C)], dst_v)
        pltpu.async_copy(table.at[src_v], rows_v, sem).wait()
        pltpu.sync_copy(rows_v, acc.at[dst_v], add=True)
        return carry

    lax.fori_loop(0, CHUNKS, chunk, 0)
    plsc.subcore_barrier()
    for off, sz in _PIECES:
        pltpu.sync_copy(acc.at[pl.ds(row0 + off, sz)], rows_v.at[pl.ds(0, sz)])
        pltpu.sync_copy(rows_v.at[pl.ds(0, sz)],
                        sums_out.at[c, pl.ds(row0 + off, sz)])


@functools.lru_cache(maxsize=None)
def _get_seg_kernel():
    return pl.kernel(
        _seg_body,
        out_type=jax.ShapeDtypeStruct((NC, NPAD, WD), jnp.float32),
        mesh=plsc.VectorSubcoreMesh(core_axis_name="c", subcore_axis_name="s",
                                    num_cores=NC, num_subcores=NS),
        scratch_types=(
            pltpu.VMEM((C,), jnp.int32),
            pltpu.VMEM((C,), jnp.int32),
            pltpu.VMEM((C, WD), jnp.float32),
            pltpu.VMEM_SHARED((NPAD, WD), jnp.float32),
            pltpu.SemaphoreType.DMA,
        ),
        compiler_params=pltpu.CompilerParams(use_tc_tiling_on_sc=False),
    )


def _dotT(a, w):
    return lax.dot_general(a, w, (((1,), (1,)), ((), ())),
                           preferred_element_type=jnp.float32)


def _layer0_body(x_ref, sums_ref,
                 wl_i, bl_i, wr_i, wl_r, bl_r, wr_r, out_ref):
    x_b = x_ref[...]
    ones = jnp.ones((BLK, 1), jnp.float32)
    zeros = jnp.zeros((BLK, WD - D - 1), jnp.float32)
    for r, (wl, bl, wr) in enumerate(((wl_i, bl_i, wr_i), (wl_r, bl_r, wr_r))):
        blk = sums_ref[r]
        cnt = jnp.maximum(blk[:, D:D + 1], 1.0)
        mean = blk[:, 0:D] / cnt
        h = _dotT(mean, wl[...]) + bl[...] + _dotT(x_b, wr[...])
        h = jnp.maximum(h, 0.0)
        out_ref[r] = jnp.concatenate([h, ones, zeros], axis=1)


def _layer1_body(h_ref, sums_ref,
                 wl_i, bl_i, wr_i, wl_r, bl_r, wr_r, att_ref, out_ref):
    a = att_ref[...]
    e = jnp.exp(a - jnp.max(a))
    w = e / jnp.sum(e)
    outs = []
    for r, (wl, bl, wr) in enumerate(((wl_i, bl_i, wr_i), (wl_r, bl_r, wr_r))):
        blk = sums_ref[r]
        cnt = jnp.maximum(blk[:, D:D + 1], 1.0)
        mean = blk[:, 0:D] / cnt
        outs.append(_dotT(mean, wl[...]) + bl[...]
                    + _dotT(h_ref[r][:, 0:D], wr[...]))
    comb = w[0:1, 0:1] * outs[0] + w[0:1, 1:2] * outs[1]
    nrm = jnp.sqrt(jnp.sum(comb * comb, axis=1, keepdims=True))
    out_ref[...] = comb / jnp.maximum(nrm, 1e-12)


def _full_spec(shape):
    return pl.BlockSpec(shape, lambda i: tuple(0 for _ in shape))


def _tc_layer0(x, sums, wl_i, bl_i, wr_i, wl_r, bl_r, wr_r):
    return pl.pallas_call(
        _layer0_body,
        grid=(N // BLK,),
        in_specs=[
            pl.BlockSpec((BLK, D), lambda i: (i, 0)),
            pl.BlockSpec((2, BLK, WD), lambda i: (0, i, 0)),
            _full_spec((D, D)), _full_spec((1, D)), _full_spec((D, D)),
            _full_spec((D, D)), _full_spec((1, D)), _full_spec((D, D)),
        ],
        out_specs=pl.BlockSpec((2, BLK, WD), lambda i: (0, i, 0)),
        out_shape=jax.ShapeDtypeStruct((2, N, WD), jnp.float32),
    )(x, sums, wl_i, bl_i, wr_i, wl_r, bl_r, wr_r)


def _tc_layer1(h, sums, wl_i, bl_i, wr_i, wl_r, bl_r, wr_r, att):
    return pl.pallas_call(
        _layer1_body,
        grid=(N // BLK,),
        in_specs=[
            pl.BlockSpec((2, BLK, WD), lambda i: (0, i, 0)),
            pl.BlockSpec((2, BLK, WD), lambda i: (0, i, 0)),
            _full_spec((D, D)), _full_spec((1, D)), _full_spec((D, D)),
            _full_spec((D, D)), _full_spec((1, D)), _full_spec((D, D)),
            _full_spec((1, 2)),
        ],
        out_specs=pl.BlockSpec((BLK, D), lambda i: (i, 0)),
        out_shape=jax.ShapeDtypeStruct((N, D), jnp.float32),
    )(h, sums, wl_i, bl_i, wr_i, wl_r, bl_r, wr_r, att)


def kernel(x, edge_index_imports, edge_index_references,
           Wl0_imports, bl0_imports, Wr0_imports,
           Wl1_imports, bl1_imports, Wr1_imports,
           Wl0_references, bl0_references, Wr0_references,
           Wl1_references, bl1_references, Wr1_references,
           edge_type_attention):
    src_i, dst_i = edge_index_imports[0], edge_index_imports[1]
    src_r, dst_r = edge_index_references[0], edge_index_references[1]
    pad = EPAD - E

    def padv(a, v):
        return jnp.concatenate([a, jnp.full((pad,), v, jnp.int32)])

    src0 = jnp.concatenate([padv(src_i, 0), padv(src_r, 0)])
    src1 = jnp.concatenate([padv(src_i, 0), padv(src_r + N, 0)])
    dsts = jnp.concatenate([padv(dst_i, PADROW), padv(dst_r, PADROW)])
    z2d = jnp.zeros((C, WD), jnp.float32)

    t0 = jnp.concatenate(
        [x, jnp.ones((N, 1), jnp.float32),
         jnp.zeros((N, WD - D - 1), jnp.float32)], axis=1)

    seg = _get_seg_kernel()
    sums0 = seg(t0, src0, dsts, z2d)
    h = _tc_layer0(x, sums0,
                   Wl0_imports, bl0_imports.reshape(1, D), Wr0_imports,
                   Wl0_references, bl0_references.reshape(1, D),
                   Wr0_references)
    sums1 = seg(h.reshape(2 * N, WD), src1, dsts, z2d)
    return _tc_layer1(h, sums1,
                      Wl1_imports, bl1_imports.reshape(1, D), Wr1_imports,
                      Wl1_references, bl1_references.reshape(1, D),
                      Wr1_references,
                      edge_type_attention.reshape(1, 2))


# speedup vs baseline: 4.0188x; 1.0582x over previous
import functools

import jax
import jax.numpy as jnp
from jax import lax
from jax.experimental import pallas as pl
from jax.experimental.pallas import tpu as pltpu
from jax.experimental.pallas import tpu_sc as plsc

N = 10000
E = 320000
D = 128
WD = 144

NC = 2
NS = 16
C = 128
EPT = 20096
EPAD = EPT * NS
CHUNKS = EPT // C
NPAD = 10112
RPT = NPAD // NS
PADROW = 10048

BLK = 400


_PIECES = []
_off = 0
while _off < RPT:
    _PIECES.append((_off, min(C, RPT - _off)))
    _off += C


def _seg_body(table, src, dst, z2d,
              sums_out,
              src_v, dst_v, rows_v, acc, sem):
    c = lax.axis_index("c")
    s = lax.axis_index("s")
    row0 = s * RPT
    pltpu.sync_copy(z2d, rows_v)
    for off, sz in _PIECES:
        pltpu.sync_copy(rows_v.at[pl.ds(0, sz)], acc.at[pl.ds(row0 + off, sz)])
    plsc.subcore_barrier()

    ebase = s * EPT

    def chunk(i, carry):
        base = c * EPAD + ebase + i * C
        pltpu.sync_copy(src.at[pl.ds(base, C)], src_v)
        pltpu.sync_copy(dst.at[pl.ds(base, C)], dst_v)
        pltpu.async_copy(table.at[src_v], rows_v, sem).wait()
        pltpu.sync_copy(rows_v, acc.at[dst_v], add=True)
        return carry

    lax.fori_loop(0, CHUNKS, chunk, 0)
    plsc.subcore_barrier()
    for off, sz in _PIECES:
        pltpu.sync_copy(acc.at[pl.ds(row0 + off, sz)], rows_v.at[pl.ds(0, sz)])
        pltpu.sync_copy(rows_v.at[pl.ds(0, sz)],
                        sums_out.at[c, pl.ds(row0 + off, sz)])


@functools.lru_cache(maxsize=None)
def _get_seg_kernel():
    return pl.kernel(
        _seg_body,
        out_type=jax.ShapeDtypeStruct((NC, NPAD, WD), jnp.float32),
        mesh=plsc.VectorSubcoreMesh(core_axis_name="c", subcore_axis_name="s",
                                    num_cores=NC, num_subcores=NS),
        scratch_types=(
            pltpu.VMEM((C,), jnp.int32),
            pltpu.VMEM((C,), jnp.int32),
            pltpu.VMEM((C, WD), jnp.float32),
            pltpu.VMEM_SHARED((NPAD, WD), jnp.float32),
            pltpu.SemaphoreType.DMA,
        ),
        compiler_params=pltpu.CompilerParams(use_tc_tiling_on_sc=False),
    )


def _dotT(a, w):
    return lax.dot_general(a, w, (((1,), (1,)), ((), ())),
                           preferred_element_type=jnp.float32)


def _layer0_body(x_ref, sums_ref,
                 wl_i, bl_i, wr_i, wl_r, bl_r, wr_r, out_ref):
    x_b = x_ref[...]
    ones = jnp.ones((BLK, 1), jnp.float32)
    zeros = jnp.zeros((BLK, WD - D - 1), jnp.float32)
    for r, (wl, bl, wr) in enumerate(((wl_i, bl_i, wr_i), (wl_r, bl_r, wr_r))):
        blk = sums_ref[r]
        cnt = jnp.maximum(blk[:, D:D + 1], 1.0)
        mean = blk[:, 0:D] / cnt
        h = _dotT(mean, wl[...]) + bl[...] + _dotT(x_b, wr[...])
        h = jnp.maximum(h, 0.0)
        out_ref[r] = jnp.concatenate([h, ones, zeros], axis=1)


def _layer1_body(h_ref, sums_ref,
                 wl_i, bl_i, wr_i, wl_r, bl_r, wr_r, att_ref, out_ref):
    a = att_ref[...]
    e = jnp.exp(a - jnp.max(a))
    w = e / jnp.sum(e)
    outs = []
    for r, (wl, bl, wr) in enumerate(((wl_i, bl_i, wr_i), (wl_r, bl_r, wr_r))):
        blk = sums_ref[r]
        cnt = jnp.maximum(blk[:, D:D + 1], 1.0)
        mean = blk[:, 0:D] / cnt
        outs.append(_dotT(mean, wl[...]) + bl[...]
                    + _dotT(h_ref[r][:, 0:D], wr[...]))
    comb = w[0:1, 0:1] * outs[0] + w[0:1, 1:2] * outs[1]
    nrm = jnp.sqrt(jnp.sum(comb * comb, axis=1, keepdims=True))
    out_ref[...] = comb / jnp.maximum(nrm, 1e-12)


def _full_spec(shape):
    return pl.BlockSpec(shape, lambda i: tuple(0 for _ in shape))


def _tc_layer0(x, sums, wl_i, bl_i, wr_i, wl_r, bl_r, wr_r):
    return pl.pallas_call(
        _layer0_body,
        grid=(N // BLK,),
        in_specs=[
            pl.BlockSpec((BLK, D), lambda i: (i, 0)),
            pl.BlockSpec((2, BLK, WD), lambda i: (0, i, 0)),
            _full_spec((D, D)), _full_spec((1, D)), _full_spec((D, D)),
            _full_spec((D, D)), _full_spec((1, D)), _full_spec((D, D)),
        ],
        out_specs=pl.BlockSpec((2, BLK, WD), lambda i: (0, i, 0)),
        out_shape=jax.ShapeDtypeStruct((2, N, WD), jnp.float32),
    )(x, sums, wl_i, bl_i, wr_i, wl_r, bl_r, wr_r)


def _tc_layer1(h, sums, wl_i, bl_i, wr_i, wl_r, bl_r, wr_r, att):
    return pl.pallas_call(
        _layer1_body,
        grid=(N // BLK,),
        in_specs=[
            pl.BlockSpec((2, BLK, WD), lambda i: (0, i, 0)),
            pl.BlockSpec((2, BLK, WD), lambda i: (0, i, 0)),
            _full_spec((D, D)), _full_spec((1, D)), _full_spec((D, D)),
            _full_spec((D, D)), _full_spec((1, D)), _full_spec((D, D)),
            _full_spec((1, 2)),
        ],
        out_specs=pl.BlockSpec((BLK, D), lambda i: (i, 0)),
        out_shape=jax.ShapeDtypeStruct((N, D), jnp.float32),
    )(h, sums, wl_i, bl_i, wr_i, wl_r, bl_r, wr_r, att)


def kernel(x, edge_index_imports, edge_index_references,
           Wl0_imports, bl0_imports, Wr0_imports,
           Wl1_imports, bl1_imports, Wr1_imports,
           Wl0_references, bl0_references, Wr0_references,
           Wl1_references, bl1_references, Wr1_references,
           edge_type_attention):
    src_i, dst_i = edge_index_imports[0], edge_index_imports[1]
    src_r, dst_r = edge_index_references[0], edge_index_references[1]
    pad = EPAD - E

    def padv(a, v):
        return jnp.concatenate([a, jnp.full((pad,), v, jnp.int32)])

    src0 = jnp.concatenate([padv(src_i, 0), padv(src_r, 0)])
    src1 = jnp.concatenate([padv(src_i, 0), padv(src_r + N, 0)])
    dsts = jnp.concatenate([padv(dst_i, PADROW), padv(dst_r, PADROW)])
    z2d = jnp.zeros((C, WD), jnp.float32)

    t0 = jnp.concatenate(
        [x, jnp.ones((N, 1), jnp.float32),
         jnp.zeros((N, WD - D - 1), jnp.float32)], axis=1)

    seg = _get_seg_kernel()
    sums0 = seg(t0, src0, dsts, z2d)
    h = _tc_layer0(x, sums0,
                   Wl0_imports, bl0_imports.reshape(1, D), Wr0_imports,
                   Wl0_references, bl0_references.reshape(1, D),
                   Wr0_references)
    sums1 = seg(h.reshape(2 * N, WD), src1, dsts, z2d)
    return _tc_layer1(h, sums1,
                      Wl1_imports, bl1_imports.reshape(1, D), Wr1_imports,
                      Wl1_references, bl1_references.reshape(1, D),
                      Wr1_references,
                      edge_type_attention.reshape(1, 2))


# speedup vs baseline: 4.4094x; 1.0972x over previous
import functools

import jax
import jax.numpy as jnp
from jax import lax
from jax.experimental import pallas as pl
from jax.experimental.pallas import tpu as pltpu
from jax.experimental.pallas import tpu_sc as plsc

N = 10000
E = 320000
D = 128
WD = 144

NC = 2
NS = 16
C = 128
EPT = 20096
EPAD = EPT * NS
CHUNKS = EPT // C
TOTCH = 2 * EPAD // C
NPAD = 10112
RPT = NPAD // NS
PADROW = 10048

BLK = 400


_PIECES = []
_off = 0
while _off < RPT:
    _PIECES.append((_off, min(C, RPT - _off)))
    _off += C


def _seg_body(table, sd, z2d,
              sums_out,
              sdb, rows_v, acc, sem):
    c = lax.axis_index("c")
    s = lax.axis_index("s")
    row0 = s * RPT
    pltpu.sync_copy(z2d, rows_v)
    for off, sz in _PIECES:
        pltpu.sync_copy(rows_v.at[pl.ds(0, sz)], acc.at[pl.ds(row0 + off, sz)])
    plsc.subcore_barrier()

    tc0 = c * (TOTCH // 2) + s * CHUNKS

    def chunk(i, carry):
        pltpu.sync_copy(sd.at[tc0 + i], sdb)
        pltpu.async_copy(table.at[sdb.at[0]], rows_v, sem).wait()
        pltpu.sync_copy(rows_v, acc.at[sdb.at[1]], add=True)
        return carry

    lax.fori_loop(0, CHUNKS, chunk, 0)
    plsc.subcore_barrier()
    for off, sz in _PIECES:
        pltpu.sync_copy(acc.at[pl.ds(row0 + off, sz)], rows_v.at[pl.ds(0, sz)])
        pltpu.sync_copy(rows_v.at[pl.ds(0, sz)],
                        sums_out.at[c, pl.ds(row0 + off, sz)])


@functools.lru_cache(maxsize=None)
def _get_seg_kernel():
    return pl.kernel(
        _seg_body,
        out_type=jax.ShapeDtypeStruct((NC, NPAD, WD), jnp.float32),
        mesh=plsc.VectorSubcoreMesh(core_axis_name="c", subcore_axis_name="s",
                                    num_cores=NC, num_subcores=NS),
        scratch_types=(
            pltpu.VMEM((2, C), jnp.int32),
            pltpu.VMEM((C, WD), jnp.float32),
            pltpu.VMEM_SHARED((NPAD, WD), jnp.float32),
            pltpu.SemaphoreType.DMA,
        ),
        compiler_params=pltpu.CompilerParams(use_tc_tiling_on_sc=False),
    )


def _dotT(a, w):
    return lax.dot_general(a, w, (((1,), (1,)), ((), ())),
                           preferred_element_type=jnp.float32)


def _layer0_body(x_ref, sums_ref,
                 wl_i, bl_i, wr_i, wl_r, bl_r, wr_r, out_ref):
    x_b = x_ref[...]
    ones = jnp.ones((BLK, 1), jnp.float32)
    zeros = jnp.zeros((BLK, WD - D - 1), jnp.float32)
    for r, (wl, bl, wr) in enumerate(((wl_i, bl_i, wr_i), (wl_r, bl_r, wr_r))):
        blk = sums_ref[r]
        cnt = jnp.maximum(blk[:, D:D + 1], 1.0)
        mean = blk[:, 0:D] / cnt
        h = _dotT(mean, wl[...]) + bl[...] + _dotT(x_b, wr[...])
        h = jnp.maximum(h, 0.0)
        out_ref[r] = jnp.concatenate([h, ones, zeros], axis=1)


def _layer1_body(h_ref, sums_ref,
                 wl_i, bl_i, wr_i, wl_r, bl_r, wr_r, att_ref, out_ref):
    a = att_ref[...]
    e = jnp.exp(a - jnp.max(a))
    w = e / jnp.sum(e)
    outs = []
    for r, (wl, bl, wr) in enumerate(((wl_i, bl_i, wr_i), (wl_r, bl_r, wr_r))):
        blk = sums_ref[r]
        cnt = jnp.maximum(blk[:, D:D + 1], 1.0)
        mean = blk[:, 0:D] / cnt
        outs.append(_dotT(mean, wl[...]) + bl[...]
                    + _dotT(h_ref[r][:, 0:D], wr[...]))
    comb = w[0:1, 0:1] * outs[0] + w[0:1, 1:2] * outs[1]
    nrm = jnp.sqrt(jnp.sum(comb * comb, axis=1, keepdims=True))
    out_ref[...] = comb / jnp.maximum(nrm, 1e-12)


def _full_spec(shape):
    return pl.BlockSpec(shape, lambda i: tuple(0 for _ in shape))


def _tc_layer0(x, sums, wl_i, bl_i, wr_i, wl_r, bl_r, wr_r):
    return pl.pallas_call(
        _layer0_body,
        grid=(N // BLK,),
        in_specs=[
            pl.BlockSpec((BLK, D), lambda i: (i, 0)),
            pl.BlockSpec((2, BLK, WD), lambda i: (0, i, 0)),
            _full_spec((D, D)), _full_spec((1, D)), _full_spec((D, D)),
            _full_spec((D, D)), _full_spec((1, D)), _full_spec((D, D)),
        ],
        out_specs=pl.BlockSpec((2, BLK, WD), lambda i: (0, i, 0)),
        out_shape=jax.ShapeDtypeStruct((2, N, WD), jnp.float32),
    )(x, sums, wl_i, bl_i, wr_i, wl_r, bl_r, wr_r)


def _tc_layer1(h, sums, wl_i, bl_i, wr_i, wl_r, bl_r, wr_r, att):
    return pl.pallas_call(
        _layer1_body,
        grid=(N // BLK,),
        in_specs=[
            pl.BlockSpec((2, BLK, WD), lambda i: (0, i, 0)),
            pl.BlockSpec((2, BLK, WD), lambda i: (0, i, 0)),
            _full_spec((D, D)), _full_spec((1, D)), _full_spec((D, D)),
            _full_spec((D, D)), _full_spec((1, D)), _full_spec((D, D)),
            _full_spec((1, 2)),
        ],
        out_specs=pl.BlockSpec((BLK, D), lambda i: (i, 0)),
        out_shape=jax.ShapeDtypeStruct((N, D), jnp.float32),
    )(h, sums, wl_i, bl_i, wr_i, wl_r, bl_r, wr_r, att)


def kernel(x, edge_index_imports, edge_index_references,
           Wl0_imports, bl0_imports, Wr0_imports,
           Wl1_imports, bl1_imports, Wr1_imports,
           Wl0_references, bl0_references, Wr0_references,
           Wl1_references, bl1_references, Wr1_references,
           edge_type_attention):
    src_i, dst_i = edge_index_imports[0], edge_index_imports[1]
    src_r, dst_r = edge_index_references[0], edge_index_references[1]
    pad = EPAD - E

    def padv(a, v):
        return jnp.concatenate([a, jnp.full((pad,), v, jnp.int32)])

    dch = jnp.concatenate([padv(dst_i, PADROW),
                           padv(dst_r, PADROW)]).reshape(TOTCH, C)

    def pack(src_flat):
        return jnp.stack([src_flat.reshape(TOTCH, C), dch], axis=1)

    sd0 = pack(jnp.concatenate([padv(src_i, 0), padv(src_r, 0)]))
    sd1 = pack(jnp.concatenate([padv(src_i, 0), padv(src_r + N, 0)]))
    z2d = jnp.zeros((C, WD), jnp.float32)

    t0 = jnp.concatenate(
        [x, jnp.ones((N, 1), jnp.float32),
         jnp.zeros((N, WD - D - 1), jnp.float32)], axis=1)

    seg = _get_seg_kernel()
    sums0 = seg(t0, sd0, z2d)
    h = _tc_layer0(x, sums0,
                   Wl0_imports, bl0_imports.reshape(1, D), Wr0_imports,
                   Wl0_references, bl0_references.reshape(1, D),
                   Wr0_references)
    sums1 = seg(h.reshape(2 * N, WD), sd1, z2d)
    return _tc_layer1(h, sums1,
                      Wl1_imports, bl1_imports.reshape(1, D), Wr1_imports,
                      Wl1_references, bl1_references.reshape(1, D),
                      Wr1_references,
                      edge_type_attention.reshape(1, 2))


# speedup vs baseline: 5.1353x; 1.1646x over previous
import functools

import jax
import jax.numpy as jnp
from jax import lax
from jax.experimental import pallas as pl
from jax.experimental.pallas import tpu as pltpu
from jax.experimental.pallas import tpu_sc as plsc

N = 10000
E = 320000
D = 128
WD = 144

NC = 2
NS = 16
C = 128
EPT = 20096
EPAD = EPT * NS
CHUNKS = EPT // C
TOTCH = 2 * EPAD // C
NPAD = 10112
RPT = NPAD // NS
PADROW = 10048

BLK = 400


_PIECES = []
_off = 0
while _off < RPT:
    _PIECES.append((_off, min(C, RPT - _off)))
    _off += C


def _seg_body(table, sd, z2d,
              sums_out,
              sdb, rows_v, rows_w, acc, sem, sem2):
    c = lax.axis_index("c")
    s = lax.axis_index("s")
    row0 = s * RPT
    pltpu.sync_copy(z2d, rows_v)
    for off, sz in _PIECES:
        pltpu.sync_copy(rows_v.at[pl.ds(0, sz)], acc.at[pl.ds(row0 + off, sz)])
    plsc.subcore_barrier()

    tc0 = c * (TOTCH // 2) + s * CHUNKS

    def pair(i, carry):
        pltpu.sync_copy(sd.at[pl.ds(tc0 + 2 * i, 2)], sdb)
        d0 = pltpu.async_copy(table.at[sdb.at[0, 0]], rows_v, sem)
        d1 = pltpu.async_copy(table.at[sdb.at[1, 0]], rows_w, sem2)
        d0.wait()
        pltpu.sync_copy(rows_v, acc.at[sdb.at[0, 1]], add=True)
        d1.wait()
        pltpu.sync_copy(rows_w, acc.at[sdb.at[1, 1]], add=True)
        return carry

    lax.fori_loop(0, CHUNKS // 2, pair, 0)
    plsc.subcore_barrier()
    for off, sz in _PIECES:
        pltpu.sync_copy(acc.at[pl.ds(row0 + off, sz)], rows_v.at[pl.ds(0, sz)])
        pltpu.sync_copy(rows_v.at[pl.ds(0, sz)],
                        sums_out.at[c, pl.ds(row0 + off, sz)])


@functools.lru_cache(maxsize=None)
def _get_seg_kernel():
    return pl.kernel(
        _seg_body,
        out_type=jax.ShapeDtypeStruct((NC, NPAD, WD), jnp.float32),
        mesh=plsc.VectorSubcoreMesh(core_axis_name="c", subcore_axis_name="s",
                                    num_cores=NC, num_subcores=NS),
        scratch_types=(
            pltpu.VMEM((2, 2, C), jnp.int32),
            pltpu.VMEM((C, WD), jnp.float32),
            pltpu.VMEM((C, WD), jnp.float32),
            pltpu.VMEM_SHARED((NPAD, WD), jnp.float32),
            pltpu.SemaphoreType.DMA,
            pltpu.SemaphoreType.DMA,
        ),
        compiler_params=pltpu.CompilerParams(use_tc_tiling_on_sc=False),
    )


def _dotT(a, w):
    return lax.dot_general(a, w, (((1,), (1,)), ((), ())),
                           preferred_element_type=jnp.float32)


def _layer0_body(x_ref, sums_ref,
                 wl_i, bl_i, wr_i, wl_r, bl_r, wr_r, out_ref):
    x_b = x_ref[...]
    ones = jnp.ones((BLK, 1), jnp.float32)
    zeros = jnp.zeros((BLK, WD - D - 1), jnp.float32)
    for r, (wl, bl, wr) in enumerate(((wl_i, bl_i, wr_i), (wl_r, bl_r, wr_r))):
        blk = sums_ref[r]
        cnt = jnp.maximum(blk[:, D:D + 1], 1.0)
        mean = blk[:, 0:D] / cnt
        h = _dotT(mean, wl[...]) + bl[...] + _dotT(x_b, wr[...])
        h = jnp.maximum(h, 0.0)
        out_ref[r] = jnp.concatenate([h, ones, zeros], axis=1)


def _layer1_body(h_ref, sums_ref,
                 wl_i, bl_i, wr_i, wl_r, bl_r, wr_r, att_ref, out_ref):
    a = att_ref[...]
    e = jnp.exp(a - jnp.max(a))
    w = e / jnp.sum(e)
    outs = []
    for r, (wl, bl, wr) in enumerate(((wl_i, bl_i, wr_i), (wl_r, bl_r, wr_r))):
        blk = sums_ref[r]
        cnt = jnp.maximum(blk[:, D:D + 1], 1.0)
        mean = blk[:, 0:D] / cnt
        outs.append(_dotT(mean, wl[...]) + bl[...]
                    + _dotT(h_ref[r][:, 0:D], wr[...]))
    comb = w[0:1, 0:1] * outs[0] + w[0:1, 1:2] * outs[1]
    nrm = jnp.sqrt(jnp.sum(comb * comb, axis=1, keepdims=True))
    out_ref[...] = comb / jnp.maximum(nrm, 1e-12)


def _full_spec(shape):
    return pl.BlockSpec(shape, lambda i: tuple(0 for _ in shape))


def _tc_layer0(x, sums, wl_i, bl_i, wr_i, wl_r, bl_r, wr_r):
    return pl.pallas_call(
        _layer0_body,
        grid=(N // BLK,),
        in_specs=[
            pl.BlockSpec((BLK, D), lambda i: (i, 0)),
            pl.BlockSpec((2, BLK, WD), lambda i: (0, i, 0)),
            _full_spec((D, D)), _full_spec((1, D)), _full_spec((D, D)),
            _full_spec((D, D)), _full_spec((1, D)), _full_spec((D, D)),
        ],
        out_specs=pl.BlockSpec((2, BLK, WD), lambda i: (0, i, 0)),
        out_shape=jax.ShapeDtypeStruct((2, N, WD), jnp.float32),
    )(x, sums, wl_i, bl_i, wr_i, wl_r, bl_r, wr_r)


def _tc_layer1(h, sums, wl_i, bl_i, wr_i, wl_r, bl_r, wr_r, att):
    return pl.pallas_call(
        _layer1_body,
        grid=(N // BLK,),
        in_specs=[
            pl.BlockSpec((2, BLK, WD), lambda i: (0, i, 0)),
            pl.BlockSpec((2, BLK, WD), lambda i: (0, i, 0)),
            _full_spec((D, D)), _full_spec((1, D)), _full_spec((D, D)),
            _full_spec((D, D)), _full_spec((1, D)), _full_spec((D, D)),
            _full_spec((1, 2)),
        ],
        out_specs=pl.BlockSpec((BLK, D), lambda i: (i, 0)),
        out_shape=jax.ShapeDtypeStruct((N, D), jnp.float32),
    )(h, sums, wl_i, bl_i, wr_i, wl_r, bl_r, wr_r, att)


def kernel(x, edge_index_imports, edge_index_references,
           Wl0_imports, bl0_imports, Wr0_imports,
           Wl1_imports, bl1_imports, Wr1_imports,
           Wl0_references, bl0_references, Wr0_references,
           Wl1_references, bl1_references, Wr1_references,
           edge_type_attention):
    src_i, dst_i = edge_index_imports[0], edge_index_imports[1]
    src_r, dst_r = edge_index_references[0], edge_index_references[1]
    pad = EPAD - E

    def padv(a, v):
        return jnp.concatenate([a, jnp.full((pad,), v, jnp.int32)])

    dch = jnp.concatenate([padv(dst_i, PADROW),
                           padv(dst_r, PADROW)]).reshape(TOTCH, C)

    def pack(src_flat):
        return jnp.stack([src_flat.reshape(TOTCH, C), dch], axis=1)

    sd0 = pack(jnp.concatenate([padv(src_i, 0), padv(src_r, 0)]))
    sd1 = pack(jnp.concatenate([padv(src_i, 0), padv(src_r + N, 0)]))
    z2d = jnp.zeros((C, WD), jnp.float32)

    t0 = jnp.concatenate(
        [x, jnp.ones((N, 1), jnp.float32),
         jnp.zeros((N, WD - D - 1), jnp.float32)], axis=1)

    seg = _get_seg_kernel()
    sums0 = seg(t0, sd0, z2d)
    h = _tc_layer0(x, sums0,
                   Wl0_imports, bl0_imports.reshape(1, D), Wr0_imports,
                   Wl0_references, bl0_references.reshape(1, D),
                   Wr0_references)
    sums1 = seg(h.reshape(2 * N, WD), sd1, z2d)
    return _tc_layer1(h, sums1,
                      Wl1_imports, bl1_imports.reshape(1, D), Wr1_imports,
                      Wl1_references, bl1_references.reshape(1, D),
                      Wr1_references,
                      edge_type_attention.reshape(1, 2))
